# Initial kernel scaffold; baseline (speedup 1.0000x reference)
import jax
import jax.numpy as jnp
from jax.experimental import pallas as pl


def kernel(x, gate_w, centroids):
    raise NotImplementedError("write your pallas kernel here")


# speedup vs baseline: 1.3328x; 1.3328x over previous
import jax
import jax.numpy as jnp
from jax.experimental import pallas as pl
from jax.experimental.pallas import tpu as pltpu

N_EXP = 16
TOPK = 2
EPS = 0.05
ITERS = 20
BM = 512


def _router_kernel(x_ref, w_ref, pit_ref, idx_ref, loss_ref, acc_ref):
    t = pl.program_id(0)
    nt = pl.num_programs(0)
    xb = x_ref[...]
    w = w_ref[...]
    lg = jax.lax.dot_general(
        w, xb, (((1,), (1,)), ((), ())), preferred_element_type=jnp.float32
    )
    acc_ref[:, pl.ds(t * BM, BM)] = lg

    @pl.when(t == nt - 1)
    def _finalize():
        la0 = acc_ref[...] * (-1.0 / EPS)

        def body(_, la):
            m0 = jnp.max(la, axis=0, keepdims=True)
            la = la - (m0 + jnp.log(jnp.sum(jnp.exp(la - m0), axis=0, keepdims=True)))
            m1 = jnp.max(la, axis=1, keepdims=True)
            la = la - (m1 + jnp.log(jnp.sum(jnp.exp(la - m1), axis=1, keepdims=True)))
            return la

        la = jax.lax.fori_loop(0, ITERS, body, la0)
        pit = jnp.exp(la)
        pit_ref[...] = pit

        iota = jax.lax.broadcasted_iota(jnp.int32, pit.shape, 0)
        mx1 = jnp.max(pit, axis=0, keepdims=True)
        i1 = jnp.min(jnp.where(pit == mx1, iota, N_EXP), axis=0, keepdims=True)
        masked = jnp.where(iota == i1, -jnp.inf, pit)
        mx2 = jnp.max(masked, axis=0, keepdims=True)
        i2 = jnp.min(jnp.where(masked == mx2, iota, N_EXP), axis=0, keepdims=True)
        idx_ref[0:1, :] = i1
        idx_ref[1:2, :] = i2

        u = 1.0 / N_EXP
        load = jnp.mean(pit, axis=1, keepdims=True)
        loss_ref[...] = jnp.sum(
            u * (jnp.log(u) - jnp.log(load)), axis=(0, 1), keepdims=True
        )


def kernel(x, gate_w, centroids):
    b, t, d = x.shape
    n = b * t
    x2 = x.reshape(n, d)
    pit, idxt, loss = pl.pallas_call(
        _router_kernel,
        grid=(n // BM,),
        in_specs=[
            pl.BlockSpec((BM, d), lambda i: (i, 0)),
            pl.BlockSpec((N_EXP, d), lambda i: (0, 0)),
        ],
        out_specs=[
            pl.BlockSpec((N_EXP, n), lambda i: (0, 0)),
            pl.BlockSpec((TOPK, n), lambda i: (0, 0)),
            pl.BlockSpec((1, 1), lambda i: (0, 0)),
        ],
        out_shape=[
            jax.ShapeDtypeStruct((N_EXP, n), jnp.float32),
            jax.ShapeDtypeStruct((TOPK, n), jnp.int32),
            jax.ShapeDtypeStruct((1, 1), jnp.float32),
        ],
        scratch_shapes=[pltpu.VMEM((N_EXP, n), jnp.float32)],
    )(x2, gate_w)
    dispatch = pit.T.reshape(b, t, N_EXP)
    indices = idxt.T.reshape(b, t, TOPK)
    load_loss = loss[0, 0]
    return dispatch, indices, load_loss


# speedup vs baseline: 1.5637x; 1.1732x over previous
import jax
import jax.numpy as jnp
from jax.experimental import pallas as pl
from jax.experimental.pallas import tpu as pltpu

N_EXP = 16
TOPK = 2
EPS = 0.05
ITERS = 20
BM = 512


def _router_kernel(x_ref, w_ref, pit_ref, idx_ref, loss_ref, acc_ref):
    t = pl.program_id(0)
    nt = pl.num_programs(0)
    xb = x_ref[...]
    w = w_ref[...]
    lg = jax.lax.dot_general(
        w, xb, (((1,), (1,)), ((), ())), preferred_element_type=jnp.float32
    )
    la = lg * (-1.0 / EPS)
    m = jnp.max(la, axis=0, keepdims=True)
    p = jnp.exp(la - m)
    p = p * (1.0 / jnp.sum(p, axis=0, keepdims=True))
    acc_ref[:, pl.ds(t * BM, BM)] = p

    @pl.when(t == nt - 1)
    def _finalize():
        p0 = acc_ref[...]
        p0 = p0 * (1.0 / jnp.sum(p0, axis=1, keepdims=True))

        def body(_, p):
            p = p * (1.0 / jnp.sum(p, axis=0, keepdims=True))
            p = p * (1.0 / jnp.sum(p, axis=1, keepdims=True))
            return p

        pit = jax.lax.fori_loop(0, ITERS - 1, body, p0)
        pit_ref[...] = pit

        iota = jax.lax.broadcasted_iota(jnp.int32, pit.shape, 0)
        mx1 = jnp.max(pit, axis=0, keepdims=True)
        i1 = jnp.min(jnp.where(pit == mx1, iota, N_EXP), axis=0, keepdims=True)
        masked = jnp.where(iota == i1, -jnp.inf, pit)
        mx2 = jnp.max(masked, axis=0, keepdims=True)
        i2 = jnp.min(jnp.where(masked == mx2, iota, N_EXP), axis=0, keepdims=True)
        idx_ref[0:1, :] = i1
        idx_ref[1:2, :] = i2

        u = 1.0 / N_EXP
        load = jnp.mean(pit, axis=1, keepdims=True)
        loss_ref[...] = jnp.sum(
            u * (jnp.log(u) - jnp.log(load)), axis=(0, 1), keepdims=True
        )


def kernel(x, gate_w, centroids):
    b, t, d = x.shape
    n = b * t
    x2 = x.reshape(n, d)
    pit, idxt, loss = pl.pallas_call(
        _router_kernel,
        grid=(n // BM,),
        in_specs=[
            pl.BlockSpec((BM, d), lambda i: (i, 0)),
            pl.BlockSpec((N_EXP, d), lambda i: (0, 0)),
        ],
        out_specs=[
            pl.BlockSpec((N_EXP, n), lambda i: (0, 0)),
            pl.BlockSpec((TOPK, n), lambda i: (0, 0)),
            pl.BlockSpec((1, 1), lambda i: (0, 0)),
        ],
        out_shape=[
            jax.ShapeDtypeStruct((N_EXP, n), jnp.float32),
            jax.ShapeDtypeStruct((TOPK, n), jnp.int32),
            jax.ShapeDtypeStruct((1, 1), jnp.float32),
        ],
        scratch_shapes=[pltpu.VMEM((N_EXP, n), jnp.float32)],
    )(x2, gate_w)
    dispatch = pit.T.reshape(b, t, N_EXP)
    indices = idxt.T.reshape(b, t, TOPK)
    load_loss = loss[0, 0]
    return dispatch, indices, load_loss
